# Initial kernel scaffold; baseline (speedup 1.0000x reference)
import jax
import jax.numpy as jnp
from jax.experimental import pallas as pl


def kernel(x, edge_index, templates, templates_features, q0, alpha0):
    raise NotImplementedError("write your pallas kernel here")


# speedup vs baseline: 4.4380x; 4.4380x over previous
import functools

import jax
import jax.numpy as jnp
from jax import lax
from jax.experimental import pallas as pl
from jax.experimental.pallas import tpu as pltpu
from jax.experimental.pallas import tpu_sc as plsc

_NC = 2
_NS = 16
_CH = 80


def _sc_segment_sums(x, src, dst):
    n, d = x.shape
    e = src.shape[0]
    nw = _NC * _NS
    ew = e // nw
    nch = ew // _CH
    cps = -(-n // (_NS * _CH))
    npad = cps * _NS * _CH
    rps = cps * _CH
    assert ew * nw == e and nch * _CH == ew
    assert ew % 8 == 0 and _CH % 8 == 0 and d % 128 == 0

    mesh = plsc.VectorSubcoreMesh(core_axis_name="c", subcore_axis_name="s")

    @functools.partial(
        pl.kernel,
        mesh=mesh,
        out_type=[
            jax.ShapeDtypeStruct((_NC, npad, d), jnp.float32),
            jax.ShapeDtypeStruct((_NC, npad, d), jnp.float32),
        ],
        scratch_types=[
            pltpu.VMEM((_CH,), jnp.int32),
            pltpu.VMEM((_CH,), jnp.int32),
            pltpu.VMEM((_CH, d), jnp.float32),
            pltpu.VMEM((_CH, d), jnp.float32),
            pltpu.VMEM_SHARED((npad, d), jnp.float32),
            pltpu.SemaphoreType.DMA,
        ],
    )
    def sc_kernel(x_hbm, src_hbm, dst_hbm, agg_out, deg_out,
                  si_v, di_v, rows_v, ones_v, agg_sh, sem):
        cid = lax.axis_index("c")
        sid = lax.axis_index("s")
        wid = sid * _NC + cid

        def _fill(i, carry):
            for j in range(d // 16):
                rows_v[i, pl.ds(j * 16, 16)] = jnp.zeros((16,), jnp.float32)
                ones_v[i, pl.ds(j * 16, 16)] = jnp.ones((16,), jnp.float32)
            return carry
        lax.fori_loop(0, _CH, _fill, 0)

        r0 = sid * rps

        for t in range(cps):
            pltpu.sync_copy(rows_v, agg_sh.at[pl.ds(r0 + t * _CH, _CH)])
        plsc.subcore_barrier()

        def _chunk1(i, carry):
            e0 = (i * nw + wid) * _CH
            pltpu.sync_copy(src_hbm.at[pl.ds(e0, _CH)], si_v)
            pltpu.sync_copy(dst_hbm.at[pl.ds(e0, _CH)], di_v)
            pltpu.async_copy(x_hbm.at[si_v], rows_v, sem).wait()
            pltpu.sync_copy(rows_v, agg_sh.at[di_v], add=True)
            return carry
        lax.fori_loop(0, nch, _chunk1, 0)

        plsc.subcore_barrier()
        pltpu.sync_copy(agg_sh.at[pl.ds(r0, rps)],
                        agg_out.at[cid, pl.ds(r0, rps)])
        plsc.subcore_barrier()

        def _refill(i, carry):
            for j in range(d // 16):
                rows_v[i, pl.ds(j * 16, 16)] = jnp.zeros((16,), jnp.float32)
            return carry
        lax.fori_loop(0, _CH, _refill, 0)

        for t in range(cps):
            pltpu.sync_copy(rows_v, agg_sh.at[pl.ds(r0 + t * _CH, _CH)])
        plsc.subcore_barrier()

        def _chunk2(i, carry):
            e0 = (i * nw + wid) * _CH
            pltpu.sync_copy(dst_hbm.at[pl.ds(e0, _CH)], di_v)
            pltpu.sync_copy(ones_v, agg_sh.at[di_v], add=True)
            return carry
        lax.fori_loop(0, nch, _chunk2, 0)

        plsc.subcore_barrier()
        pltpu.sync_copy(agg_sh.at[pl.ds(r0, rps)],
                        deg_out.at[cid, pl.ds(r0, rps)])

    return sc_kernel(x, src, dst)


def _tc_assemble(x, agg_parts, deg_parts, qft, crow, blk):
    n, d = x.shape
    npad = agg_parts.shape[1]
    grid = (n // blk,)

    def body(x_ref, a_ref, dg_ref, dgf_ref, qft_ref, c_ref, o_ref):
        inv_m = 1.0 / (jnp.max(dgf_ref[0] + dgf_ref[1]) + 1.0)
        deg = dg_ref[0, :, 0:1] + dg_ref[1, :, 0:1]
        agg = a_ref[0] + a_ref[1]
        h = (x_ref[...] + agg) / (deg + 1.0)
        hn = jnp.sum(h * h, axis=1, keepdims=True)
        z = jnp.dot(h, qft_ref[...], preferred_element_type=jnp.float32)
        c0 = c_ref[0:1, :]
        a_t = c_ref[1:2, :]
        b_t = c_ref[2:3, :]
        alpha = c_ref[3:4, 0:1]
        feat = hn - 2.0 * z + c0
        dn = deg * inv_m
        struct = a_t - 2.0 * dn * b_t + dn * dn
        dist = (1.0 - alpha) * feat + alpha * struct + 1e-8
        o_ref[...] = jnp.log(jnp.maximum(dist, 1e-8))

    return pl.pallas_call(
        body,
        grid=grid,
        in_specs=[
            pl.BlockSpec((blk, d), lambda i: (i, 0)),
            pl.BlockSpec((_NC, blk, d), lambda i: (0, i, 0)),
            pl.BlockSpec((_NC, blk, d), lambda i: (0, i, 0)),
            pl.BlockSpec((_NC, npad, d), lambda i: (0, 0, 0)),
            pl.BlockSpec((d, 128), lambda i: (0, 0)),
            pl.BlockSpec((8, 128), lambda i: (0, 0)),
        ],
        out_specs=pl.BlockSpec((blk, 128), lambda i: (i, 0)),
        out_shape=jax.ShapeDtypeStruct((n, 128), jnp.float32),
    )(x, agg_parts, deg_parts, deg_parts, qft, crow)


def kernel(x, edge_index, templates, templates_features, q0, alpha0):
    n, d = x.shape
    t_n, j_n = q0.shape

    alpha = jax.nn.sigmoid(alpha0)[0]
    q = jax.nn.softmax(q0, axis=1)
    fn = jnp.sum(templates_features ** 2, axis=2)
    c0 = jnp.sum(q * fn, axis=1)
    qf = jnp.einsum('tj,tjd->td', q, templates_features)
    a_t = jnp.einsum('tj,tl,tjl->t', q, q, templates ** 2)
    b_t = jnp.einsum('tj,tl,tjl->t', q, q, templates)

    src = edge_index[0]
    dst = edge_index[1]
    agg_parts, deg_parts = _sc_segment_sums(x, src, dst)

    qft = jnp.zeros((d, 128), jnp.float32).at[:, :t_n].set(qf.T)
    crow = (jnp.zeros((8, 128), jnp.float32)
            .at[0, :t_n].set(c0)
            .at[1, :t_n].set(a_t)
            .at[2, :t_n].set(b_t)
            .at[3, 0].set(alpha))

    out_pad = _tc_assemble(x, agg_parts, deg_parts, qft, crow, blk=1000)
    return out_pad[:, :t_n]
